# Initial kernel scaffold; baseline (speedup 1.0000x reference)
import jax
import jax.numpy as jnp
from jax.experimental import pallas as pl


def kernel(user_emb, item_emb, A_vals, A_rows, A_cols, users, items):
    raise NotImplementedError("write your pallas kernel here")


# speedup vs baseline: 3.6237x; 3.6237x over previous
import functools

import jax
import jax.numpy as jnp
from jax import lax
from jax.experimental import pallas as pl
from jax.experimental.pallas import tpu as pltpu
from jax.experimental.pallas import tpu_sc as plsc

_U = 20000
_I = 30000
_N = _U + _I
_D = 64
_HD = 32
_B = 4096
_NNZ = 800000
_LAYERS = 3

_NC = 2
_NS = 16
_L = 16

_CH = 128
_NCHUNK = -(-_NNZ // _CH)
_NCHUNK_PAD = -(-_NCHUNK // _NS) * _NS
_NNZ_PAD = _NCHUNK_PAD * _CH
_CPT = _NCHUNK_PAD // _NS

_RPT = _N // _NS
_RW = 125
_NW = _RPT // _RW

_BPT = _B // _NS
_BSUB = _BPT // _CH


def _sc_body(x0_hbm, cols_hbm, rows_hbm, vals_hbm, users_hbm, items_hbm,
             hu_hbm, hi_hbm, xs_hbm, accum_sh,
             colsb, rowsb, valsb, gbuf, wbuf, tmp,
             huacc, hiacc, uidx, sem):
    c = lax.axis_index("c")
    s = lax.axis_index("s")
    c_n = c * _N

    z16 = jnp.zeros((_L,), jnp.float32)

    def _acc_rows(dst, base, src):
        @pl.loop(0, _CH, unroll=8)
        def _(e):
            dst[base + e, pl.ds(0, _L)] = (
                dst[base + e, pl.ds(0, _L)] + src[e, pl.ds(0, _L)])
            dst[base + e, pl.ds(_L, _L)] = (
                dst[base + e, pl.ds(_L, _L)] + src[e, pl.ds(_L, _L)])

    def _copy_rows(dst, base, src):
        @pl.loop(0, _CH, unroll=8)
        def _(e):
            dst[base + e, pl.ds(0, _L)] = src[e, pl.ds(0, _L)]
            dst[base + e, pl.ds(_L, _L)] = src[e, pl.ds(_L, _L)]

    for j in range(_BSUB):
        pltpu.sync_copy(users_hbm.at[c, pl.ds(s * _BPT + j * _CH, _CH)], uidx)
        pltpu.async_copy(x0_hbm.at[uidx], tmp, sem).wait()
        _copy_rows(huacc, j * _CH, tmp)
        pltpu.sync_copy(items_hbm.at[c, pl.ds(s * _BPT + j * _CH, _CH)], uidx)
        pltpu.async_copy(x0_hbm.at[uidx], tmp, sem).wait()
        _copy_rows(hiacc, j * _CH, tmp)

    for layer in range(_LAYERS):
        src_hbm = x0_hbm if layer == 0 else xs_hbm

        @pl.loop(0, _RW)
        def _zb(e):
            wbuf[e, pl.ds(0, _L)] = z16
            wbuf[e, pl.ds(_L, _L)] = z16

        @pl.loop(0, _NW)
        def _z(k):
            pltpu.sync_copy(wbuf, accum_sh.at[pl.ds(s * _RPT + k * _RW, _RW)])

        plsc.subcore_barrier()

        @pl.loop(0, _CPT)
        def _edges(i):
            base = (s * _CPT + i) * _CH
            pltpu.sync_copy(cols_hbm.at[c, pl.ds(base, _CH)], colsb)
            pltpu.sync_copy(rows_hbm.at[pl.ds(base, _CH)], rowsb)
            pltpu.sync_copy(vals_hbm.at[pl.ds(base, _CH)], valsb)
            pltpu.async_copy(src_hbm.at[colsb], gbuf, sem).wait()

            @pl.loop(0, _CH // _L)
            def _scale(g):
                vchunk = valsb[pl.ds(g * _L, _L)]
                for k in range(_L):
                    v = lax.broadcast(vchunk[k], (_L,))
                    e = g * _L + k
                    gbuf[e, pl.ds(0, _L)] = gbuf[e, pl.ds(0, _L)] * v
                    gbuf[e, pl.ds(_L, _L)] = gbuf[e, pl.ds(_L, _L)] * v

            pltpu.sync_copy(gbuf, accum_sh.at[rowsb], add=True)

        plsc.subcore_barrier()

        @pl.loop(0, _NW)
        def _wb(k):
            r = s * _RPT + k * _RW
            pltpu.sync_copy(accum_sh.at[pl.ds(r, _RW)], wbuf)
            pltpu.sync_copy(wbuf, xs_hbm.at[pl.ds(c_n + r, _RW)])

        plsc.subcore_barrier()

        for j in range(_BSUB):
            pltpu.sync_copy(users_hbm.at[c, pl.ds(s * _BPT + j * _CH, _CH)],
                            uidx)
            pltpu.async_copy(xs_hbm.at[uidx], tmp, sem).wait()
            _acc_rows(huacc, j * _CH, tmp)
            pltpu.sync_copy(items_hbm.at[c, pl.ds(s * _BPT + j * _CH, _CH)],
                            uidx)
            pltpu.async_copy(xs_hbm.at[uidx], tmp, sem).wait()
            _acc_rows(hiacc, j * _CH, tmp)

    pltpu.sync_copy(huacc, hu_hbm.at[c, pl.ds(s * _BPT, _BPT)])
    pltpu.sync_copy(hiacc, hi_hbm.at[c, pl.ds(s * _BPT, _BPT)])


@jax.jit
def kernel(user_emb, item_emb, A_vals, A_rows, A_cols, users, items):
    x = jnp.concatenate([user_emb, item_emb], axis=0)
    x0 = jnp.concatenate([x[:, :_HD], x[:, _HD:]], axis=0)

    pad = _NNZ_PAD - _NNZ
    cols_p = jnp.pad(A_cols, (0, pad))
    rows_p = jnp.pad(A_rows, (0, pad))
    vals_p = jnp.pad(A_vals, (0, pad))
    cols2 = jnp.stack([cols_p, cols_p + _N])
    users2 = jnp.stack([users, users + _N])
    items2 = jnp.stack([_U + items, _U + items + _N])

    mesh = plsc.VectorSubcoreMesh(core_axis_name="c", subcore_axis_name="s")
    fn = pl.kernel(
        _sc_body,
        out_type=(jax.ShapeDtypeStruct((_NC, _B, _HD), jnp.float32),
                  jax.ShapeDtypeStruct((_NC, _B, _HD), jnp.float32)),
        mesh=mesh,
        compiler_params=pltpu.CompilerParams(use_tc_tiling_on_sc=False),
        scratch_types=[
            pltpu.HBM((2 * _N, _HD), jnp.float32),
            pltpu.VMEM_SHARED((_N, _HD), jnp.float32),
            pltpu.VMEM((_CH,), jnp.int32),
            pltpu.VMEM((_CH,), jnp.int32),
            pltpu.VMEM((_CH,), jnp.float32),
            pltpu.VMEM((_CH, _HD), jnp.float32),
            pltpu.VMEM((_RW, _HD), jnp.float32),
            pltpu.VMEM((_CH, _HD), jnp.float32),
            pltpu.VMEM((_BPT, _HD), jnp.float32),
            pltpu.VMEM((_BPT, _HD), jnp.float32),
            pltpu.VMEM((_CH,), jnp.int32),
            pltpu.SemaphoreType.DMA,
        ],
    )
    hu2, hi2 = fn(x0, cols2, rows_p, vals_p, users2, items2)
    hu = jnp.concatenate([hu2[0], hu2[1]], axis=1)
    hi = jnp.concatenate([hi2[0], hi2[1]], axis=1)

    def _dot_body(hu_ref, hi_ref, o_ref):
        scale = 1.0 / float((_LAYERS + 1) * (_LAYERS + 1))
        o_ref[...] = jnp.sum(hu_ref[...] * hi_ref[...], axis=1) * scale

    return pl.pallas_call(
        _dot_body,
        out_shape=jax.ShapeDtypeStruct((_B,), jnp.float32),
    )(hu, hi)


# speedup vs baseline: 10.8335x; 2.9896x over previous
import jax
import jax.numpy as jnp
from jax import lax
from jax.experimental import pallas as pl
from jax.experimental.pallas import tpu as pltpu
from jax.experimental.pallas import tpu_sc as plsc

_U = 20000
_I = 30000
_N = _U + _I
_D = 64
_HD = 32
_B = 4096
_NNZ = 800000
_LAYERS = 3

_NC = 2
_NS = 16
_L = 16

_CH = 128
_SCC = 5
_NSC = 79
_CPT = _SCC * _NSC
_NCHUNK = _CPT * _NS
_NNZ_PAD = _NCHUNK * _CH

_RPT = _N // _NS
_RW = 125
_NW = _RPT // _RW

_BPT = _B // _NS
_BSUB = _BPT // _CH


def _sc_body(x0_hbm, cols_hbm, rows_hbm, vals_hbm, users_hbm, items_hbm,
             hu_hbm, hi_hbm, xs_hbm, accum_sh,
             colsb, rowsb, valsb, gbuf, zbuf, uidx,
             isem, gsem, ssem, wsem, zsem, usem):
    c = lax.axis_index("c")
    s = lax.axis_index("s")
    c_n = c * _N
    z16 = jnp.zeros((_L,), jnp.float32)

    @pl.loop(0, _RW)
    def _zb(e):
        zbuf[e, pl.ds(0, _L)] = z16
        zbuf[e, pl.ds(_L, _L)] = z16

    @pl.loop(0, _NW)
    def _z0(k):
        pltpu.async_copy(zbuf, accum_sh.at[pl.ds(s * _RPT + k * _RW, _RW)],
                         zsem)

    @pl.loop(0, _NW)
    def _z0w(k):
        pltpu.make_async_copy(
            zbuf, accum_sh.at[pl.ds(s * _RPT + k * _RW, _RW)], zsem).wait()

    def _gather_batch(src, lidx):
        for j in range(_BSUB):
            off = s * _BPT + j * _CH
            pltpu.sync_copy(users_hbm.at[c, pl.ds(off, _CH)], uidx)
            pltpu.async_copy(src.at[uidx], gbuf.at[0], usem).wait()
            pltpu.sync_copy(gbuf.at[0], hu_hbm.at[c, lidx, pl.ds(off, _CH)])
            pltpu.sync_copy(items_hbm.at[c, pl.ds(off, _CH)], uidx)
            pltpu.async_copy(src.at[uidx], gbuf.at[0], usem).wait()
            pltpu.sync_copy(gbuf.at[0], hi_hbm.at[c, lidx, pl.ds(off, _CH)])

    _gather_batch(x0_hbm, 0)

    plsc.subcore_barrier()

    for layer in range(_LAYERS):
        src_hbm = x0_hbm if layer == 0 else xs_hbm
        tb = s * _CPT

        def _issue_idx(slot, cb):
            pltpu.async_copy(cols_hbm.at[c, pl.ds(cb, _SCC)],
                             colsb.at[slot], isem.at[slot])
            pltpu.async_copy(rows_hbm.at[pl.ds(cb, _SCC)],
                             rowsb.at[slot], isem.at[slot])
            pltpu.async_copy(vals_hbm.at[pl.ds(cb, _SCC)],
                             valsb.at[slot], isem.at[slot])

        def _wait_idx(slot, cb):
            pltpu.make_async_copy(cols_hbm.at[c, pl.ds(cb, _SCC)],
                                  colsb.at[slot], isem.at[slot]).wait()
            pltpu.make_async_copy(rows_hbm.at[pl.ds(cb, _SCC)],
                                  rowsb.at[slot], isem.at[slot]).wait()
            pltpu.make_async_copy(vals_hbm.at[pl.ds(cb, _SCC)],
                                  valsb.at[slot], isem.at[slot]).wait()

        _issue_idx(0, tb)

        @pl.loop(0, _NSC)
        def _super(isc):
            ib = lax.rem(isc, 2)
            base = tb + isc * _SCC
            _wait_idx(ib, base)

            @pl.when(isc + 1 < _NSC)
            def _():
                _issue_idx(1 - ib, base + _SCC)

            for j in range(_SCC):
                pltpu.async_copy(src_hbm.at[colsb.at[ib, j]], gbuf.at[j],
                                 gsem.at[j])
            for j in range(_SCC):
                pltpu.make_async_copy(src_hbm.at[colsb.at[ib, j]],
                                      gbuf.at[j], gsem.at[j]).wait()

                @pl.loop(0, _CH // _L)
                def _scale(g):
                    vchunk = valsb[ib, j, pl.ds(g * _L, _L)]
                    for k in range(_L):
                        v = lax.broadcast(vchunk[k], (_L,))
                        e = g * _L + k
                        gbuf[j, e, pl.ds(0, _L)] = gbuf[j, e, pl.ds(0, _L)] * v
                        gbuf[j, e, pl.ds(_L, _L)] = (
                            gbuf[j, e, pl.ds(_L, _L)] * v)

                pltpu.async_copy(gbuf.at[j], accum_sh.at[rowsb.at[ib, j]],
                                 ssem, add=True)
            for j in range(_SCC):
                pltpu.make_async_copy(gbuf.at[j], accum_sh.at[rowsb.at[ib, j]],
                                      ssem).wait()

        plsc.subcore_barrier()

        for k in range(_NW):
            b = k % 2
            r = s * _RPT + k * _RW
            if k >= 2:
                rp = s * _RPT + (k - 2) * _RW
                pltpu.make_async_copy(gbuf.at[b, pl.ds(0, _RW)],
                                      xs_hbm.at[pl.ds(c_n + rp, _RW)],
                                      wsem.at[b]).wait()
            pltpu.sync_copy(accum_sh.at[pl.ds(r, _RW)],
                            gbuf.at[b, pl.ds(0, _RW)])
            pltpu.async_copy(gbuf.at[b, pl.ds(0, _RW)],
                             xs_hbm.at[pl.ds(c_n + r, _RW)], wsem.at[b])
            pltpu.async_copy(zbuf, accum_sh.at[pl.ds(r, _RW)], zsem)
        for k in range(_NW - 2, _NW):
            b = k % 2
            r = s * _RPT + k * _RW
            pltpu.make_async_copy(gbuf.at[b, pl.ds(0, _RW)],
                                  xs_hbm.at[pl.ds(c_n + r, _RW)],
                                  wsem.at[b]).wait()

        @pl.loop(0, _NW)
        def _zw(k):
            pltpu.make_async_copy(
                zbuf, accum_sh.at[pl.ds(s * _RPT + k * _RW, _RW)], zsem).wait()

        plsc.subcore_barrier()

        _gather_batch(xs_hbm, layer + 1)


@jax.jit
def kernel(user_emb, item_emb, A_vals, A_rows, A_cols, users, items):
    x = jnp.concatenate([user_emb, item_emb], axis=0)
    x0 = jnp.concatenate([x[:, :_HD], x[:, _HD:]], axis=0)

    pad = _NNZ_PAD - _NNZ
    cols_p = jnp.pad(A_cols, (0, pad)).reshape(_NCHUNK, _CH)
    rows_p = jnp.pad(A_rows, (0, pad)).reshape(_NCHUNK, _CH)
    vals_p = jnp.pad(A_vals, (0, pad)).reshape(_NCHUNK, _CH)
    cols2 = jnp.stack([cols_p, cols_p + _N])
    users2 = jnp.stack([users, users + _N])
    items2 = jnp.stack([_U + items, _U + items + _N])

    mesh = plsc.VectorSubcoreMesh(core_axis_name="c", subcore_axis_name="s")
    fn = pl.kernel(
        _sc_body,
        out_type=(
            jax.ShapeDtypeStruct((_NC, _LAYERS + 1, _B, _HD), jnp.float32),
            jax.ShapeDtypeStruct((_NC, _LAYERS + 1, _B, _HD), jnp.float32)),
        mesh=mesh,
        compiler_params=pltpu.CompilerParams(use_tc_tiling_on_sc=False),
        scratch_types=[
            pltpu.HBM((2 * _N, _HD), jnp.float32),
            pltpu.VMEM_SHARED((_N, _HD), jnp.float32),
            pltpu.VMEM((2, _SCC, _CH), jnp.int32),
            pltpu.VMEM((2, _SCC, _CH), jnp.int32),
            pltpu.VMEM((2, _SCC, _CH), jnp.float32),
            pltpu.VMEM((_SCC, _CH, _HD), jnp.float32),
            pltpu.VMEM((_RW, _HD), jnp.float32),
            pltpu.VMEM((_CH,), jnp.int32),
            pltpu.SemaphoreType.DMA((2,)),
            pltpu.SemaphoreType.DMA((_SCC,)),
            pltpu.SemaphoreType.DMA,
            pltpu.SemaphoreType.DMA((2,)),
            pltpu.SemaphoreType.DMA,
            pltpu.SemaphoreType.DMA,
        ],
    )
    hu2, hi2 = fn(x0, cols2, rows_p, vals_p, users2, items2)

    def _dot_body(hu_ref, hi_ref, o_ref):
        scale = 1.0 / float((_LAYERS + 1) * (_LAYERS + 1))
        hu = jnp.sum(hu_ref[...], axis=1)
        hi = jnp.sum(hi_ref[...], axis=1)
        o_ref[...] = jnp.sum(hu * hi, axis=(0, 2)) * scale

    return pl.pallas_call(
        _dot_body,
        out_shape=jax.ShapeDtypeStruct((_B,), jnp.float32),
    )(hu2, hi2)
